# baseline (speedup 1.0000x reference)
import functools

import jax
import jax.numpy as jnp
from jax import lax
from jax.experimental import pallas as pl
from jax.experimental.pallas import tpu as pltpu
from jax.experimental.pallas import tpu_sc as plsc

FEATURE_NUM = 100000
FIELD_NUM = 26
LATENT_NUM = 32
BATCH = 4096

NC = 2
NS = 16
NW = NC * NS
BPW = BATCH // NW
HB = BPW // 2
LT = LATENT_NUM // 8
NF2PAIR = FIELD_NUM // 2

_mesh = plsc.VectorSubcoreMesh(
    core_axis_name="c", subcore_axis_name="s", num_cores=NC, num_subcores=NS
)


@functools.partial(
    pl.kernel,
    out_type=jax.ShapeDtypeStruct(
        (FIELD_NUM, FIELD_NUM, LT, BATCH // 128, 8, 128), jnp.float32
    ),
    mesh=_mesh,
    scratch_types=[
        pltpu.VMEM((FIELD_NUM, BPW), jnp.int32),
        pltpu.VMEM((HB, FIELD_NUM, LATENT_NUM), jnp.float32),
        pltpu.VMEM((HB, FIELD_NUM, LATENT_NUM), jnp.float32),
        pltpu.VMEM((LT, 8, HB), jnp.float32),
        pltpu.VMEM((LT, 8, HB), jnp.float32),
        pltpu.SemaphoreType.DMA,
        pltpu.SemaphoreType.DMA,
        pltpu.SemaphoreType.DMA,
        pltpu.SemaphoreType.DMA,
    ],
    compiler_params=pltpu.CompilerParams(
        use_tc_tiling_on_sc=False, needs_layout_passes=False
    ),
)
def _gather_kernel(table_hbm, xt_hbm, out_hbm, idx_t, st0, st1, ob0, ob1,
                   g0, g1, os0, os1):
    wid = lax.axis_index("s") * NC + lax.axis_index("c")
    pltpu.sync_copy(xt_hbm.at[:, pl.ds(wid * BPW, BPW)], idx_t)

    iota = lax.iota(jnp.int32, 16)

    def start_gather(f1, h, st, gsem):
        pltpu.async_copy(
            table_hbm.at[idx_t.at[f1, pl.ds(h * HB, HB)]], st, gsem
        )

    def wait_gather(st, gsem):
        pltpu.make_async_copy(table_hbm.at[pl.ds(0, HB)], st, gsem).wait()

    def wait_store(ob, osem):
        pltpu.make_async_copy(
            ob, out_hbm.at[0, 0, :, wid, :, pl.ds(0, HB)], osem
        ).wait()

    def transpose_and_store(f1, f2, h, st, ob, osem, first):
        @pl.when(jnp.logical_not(first))
        def _():
            wait_store(ob, osem)

        f2v = jnp.full((16,), f2, jnp.int32)
        for lt in range(LT):
            for ls in range(8):
                lv = jnp.full((16,), lt * 8 + ls, jnp.int32)
                for grp in range(HB // 16):
                    vec = plsc.load_gather(st, [grp * 16 + iota, f2v, lv])
                    ob[lt, ls, pl.ds(grp * 16, 16)] = vec
        pltpu.async_copy(
            ob, out_hbm.at[f1, f2, :, wid, :, pl.ds(h * HB, HB)], osem
        )

    def half_block(f1, h, st, gsem):
        wait_gather(st, gsem)

        @pl.loop(0, NF2PAIR)
        def f2pair(j):
            first0 = jnp.logical_and(jnp.logical_and(f1 == 0, h == 0), j == 0)
            first1 = jnp.logical_and(jnp.logical_and(f1 == 0, h == 0), j == 0)
            transpose_and_store(f1, j * 2, h, st, ob0, os0, first0)
            transpose_and_store(f1, j * 2 + 1, h, st, ob1, os1, first1)

    start_gather(0, 0, st0, g0)

    @pl.loop(0, FIELD_NUM)
    def f1_loop(f1):
        start_gather(f1, 1, st1, g1)
        half_block(f1, 0, st0, g0)

        @pl.when(f1 < FIELD_NUM - 1)
        def _():
            start_gather(f1 + 1, 0, st0, g0)

        half_block(f1, 1, st1, g1)

    wait_store(ob0, os0)
    wait_store(ob1, os1)


def kernel(x, weights):
    out6 = _gather_kernel(weights, x.T)
    return out6.transpose(3, 5, 0, 1, 2, 4).reshape(
        BATCH, FIELD_NUM, FIELD_NUM, LATENT_NUM
    )


# speedup vs baseline: 1.0111x; 1.0111x over previous
import functools

import jax
import jax.numpy as jnp
from jax import lax
from jax.experimental import pallas as pl
from jax.experimental.pallas import tpu as pltpu
from jax.experimental.pallas import tpu_sc as plsc

FEATURE_NUM = 100000
FIELD_NUM = 26
LATENT_NUM = 32
BATCH = 4096

NC = 2
NS = 16
NW = NC * NS
BPW = BATCH // NW
QB = BPW // 4
LT = LATENT_NUM // 8
NSTEP = FIELD_NUM * 4
NPAIR = NSTEP // 2

_mesh = plsc.VectorSubcoreMesh(
    core_axis_name="c", subcore_axis_name="s", num_cores=NC, num_subcores=NS
)


@functools.partial(
    pl.kernel,
    out_type=jax.ShapeDtypeStruct(
        (FIELD_NUM, FIELD_NUM, LT, BATCH // 128, 8, 128), jnp.float32
    ),
    mesh=_mesh,
    scratch_types=[
        pltpu.VMEM((FIELD_NUM, BPW), jnp.int32),
        pltpu.VMEM((QB, FIELD_NUM, LATENT_NUM), jnp.float32),
        pltpu.VMEM((QB, FIELD_NUM, LATENT_NUM), jnp.float32),
        pltpu.VMEM((FIELD_NUM, LT, 8, QB), jnp.float32),
        pltpu.VMEM((FIELD_NUM, LT, 8, QB), jnp.float32),
        pltpu.SemaphoreType.DMA,
        pltpu.SemaphoreType.DMA,
        pltpu.SemaphoreType.DMA,
        pltpu.SemaphoreType.DMA,
    ],
    compiler_params=pltpu.CompilerParams(
        use_tc_tiling_on_sc=False, needs_layout_passes=False
    ),
)
def _gather_kernel(table_hbm, xt_hbm, out_hbm, idx_t, st0, st1, ob0, ob1,
                   g0, g1, os0, os1):
    wid = lax.axis_index("s") * NC + lax.axis_index("c")
    pltpu.sync_copy(xt_hbm.at[:, pl.ds(wid * BPW, BPW)], idx_t)

    iota = lax.iota(jnp.int32, 16)

    def start_gather(s, st, gsem):
        f1 = lax.shift_right_logical(s, 2)
        qoff = pl.multiple_of(lax.mul(lax.rem(s, 4), QB), QB)
        pltpu.async_copy(table_hbm.at[idx_t.at[f1, pl.ds(qoff, QB)]], st, gsem)

    def wait_gather(st, gsem):
        pltpu.make_async_copy(table_hbm.at[pl.ds(0, QB)], st, gsem).wait()

    def start_store(s, ob, osem):
        f1 = lax.shift_right_logical(s, 2)
        qoff = pl.multiple_of(lax.mul(lax.rem(s, 4), QB), QB)
        pltpu.async_copy(
            ob, out_hbm.at[f1, :, :, wid, :, pl.ds(qoff, QB)], osem
        )

    def wait_store(ob, osem):
        pltpu.make_async_copy(
            ob, out_hbm.at[0, :, :, wid, :, pl.ds(0, QB)], osem
        ).wait()

    def transpose_slab(st, ob):
        @pl.loop(0, FIELD_NUM)
        def f2loop(f2):
            f2v = jnp.full((16,), f2, jnp.int32)
            for l in range(LATENT_NUM):
                lv = jnp.full((16,), l, jnp.int32)
                for g in range(QB // 16):
                    vec = plsc.load_gather(st, [g * 16 + iota, f2v, lv])
                    ob[f2, l // 8, l % 8, pl.ds(g * 16, 16)] = vec

    start_gather(0, st0, g0)

    @pl.loop(0, NPAIR)
    def pair(p):
        s0 = p * 2

        start_gather(s0 + 1, st1, g1)
        wait_gather(st0, g0)

        @pl.when(p > 0)
        def _():
            wait_store(ob0, os0)

        transpose_slab(st0, ob0)
        start_store(s0, ob0, os0)

        @pl.when(p < NPAIR - 1)
        def _():
            start_gather(s0 + 2, st0, g0)

        wait_gather(st1, g1)

        @pl.when(p > 0)
        def _():
            wait_store(ob1, os1)

        transpose_slab(st1, ob1)
        start_store(s0 + 1, ob1, os1)

    wait_store(ob0, os0)
    wait_store(ob1, os1)


def kernel(x, weights):
    out6 = _gather_kernel(weights, x.T)
    return out6.transpose(3, 5, 0, 1, 2, 4).reshape(
        BATCH, FIELD_NUM, FIELD_NUM, LATENT_NUM
    )


# speedup vs baseline: 1.2311x; 1.2175x over previous
import functools

import jax
import jax.numpy as jnp
from jax import lax
from jax.experimental import pallas as pl
from jax.experimental.pallas import tpu as pltpu
from jax.experimental.pallas import tpu_sc as plsc

FEATURE_NUM = 100000
FIELD_NUM = 26
LATENT_NUM = 32
BATCH = 4096

NC = 2
NS = 16
NW = NC * NS
BPW = BATCH // NW
QB = BPW // 4
LT = LATENT_NUM // 8
NSTEP = FIELD_NUM * 4
NPAIR = NSTEP // 2

_mesh = plsc.VectorSubcoreMesh(
    core_axis_name="c", subcore_axis_name="s", num_cores=NC, num_subcores=NS
)


@functools.partial(
    pl.kernel,
    out_type=jax.ShapeDtypeStruct(
        (FIELD_NUM, FIELD_NUM, LT, BATCH // 128, 8, 128), jnp.float32
    ),
    mesh=_mesh,
    scratch_types=[
        pltpu.VMEM((FIELD_NUM, BPW), jnp.int32),
        pltpu.VMEM((QB, FIELD_NUM, LATENT_NUM), jnp.float32),
        pltpu.VMEM((QB, FIELD_NUM, LATENT_NUM), jnp.float32),
        pltpu.VMEM((FIELD_NUM, LT, 8, QB), jnp.float32),
        pltpu.VMEM((FIELD_NUM, LT, 8, QB), jnp.float32),
        pltpu.SemaphoreType.DMA,
        pltpu.SemaphoreType.DMA,
        pltpu.SemaphoreType.DMA,
        pltpu.SemaphoreType.DMA,
    ],
    compiler_params=pltpu.CompilerParams(
        use_tc_tiling_on_sc=False, needs_layout_passes=False
    ),
)
def _gather_kernel(table_hbm, xt_hbm, out_hbm, idx_t, st0, st1, ob0, ob1,
                   g0, g1, os0, os1):
    wid = lax.axis_index("s") * NC + lax.axis_index("c")
    pltpu.sync_copy(xt_hbm.at[:, pl.ds(wid * BPW, BPW)], idx_t)

    iota = lax.iota(jnp.int32, 16)

    def start_gather(s, st, gsem):
        f1 = lax.shift_right_logical(s, 2)
        qoff = pl.multiple_of(lax.mul(lax.rem(s, 4), QB), QB)
        pltpu.async_copy(table_hbm.at[idx_t.at[f1, pl.ds(qoff, QB)]], st, gsem)

    def wait_gather(st, gsem):
        pltpu.make_async_copy(table_hbm.at[pl.ds(0, QB)], st, gsem).wait()

    def start_store(s, ob, osem):
        f1 = lax.shift_right_logical(s, 2)
        qoff = pl.multiple_of(lax.mul(lax.rem(s, 4), QB), QB)
        pltpu.async_copy(
            ob, out_hbm.at[f1, :, :, wid, :, pl.ds(qoff, QB)], osem
        )

    def wait_store(ob, osem):
        pltpu.make_async_copy(
            ob, out_hbm.at[0, :, :, wid, :, pl.ds(0, QB)], osem
        ).wait()

    def transpose_slab(st, ob):
        pairs = [
            (l, g) for l in range(LATENT_NUM) for g in range(QB // 16)
        ]

        @pl.loop(0, FIELD_NUM)
        def f2loop(f2):
            f2v = jnp.full((16,), f2, jnp.int32)
            for w0 in range(0, len(pairs), 8):
                wave = pairs[w0:w0 + 8]
                vecs = [
                    plsc.load_gather(
                        st,
                        [g * 16 + iota, f2v, jnp.full((16,), l, jnp.int32)],
                    )
                    for (l, g) in wave
                ]
                for (l, g), vec in zip(wave, vecs):
                    ob[f2, l // 8, l % 8, pl.ds(g * 16, 16)] = vec

    start_gather(0, st0, g0)

    @pl.loop(0, NPAIR)
    def pair(p):
        s0 = p * 2

        start_gather(s0 + 1, st1, g1)
        wait_gather(st0, g0)

        @pl.when(p > 0)
        def _():
            wait_store(ob0, os0)

        transpose_slab(st0, ob0)
        start_store(s0, ob0, os0)

        @pl.when(p < NPAIR - 1)
        def _():
            start_gather(s0 + 2, st0, g0)

        wait_gather(st1, g1)

        @pl.when(p > 0)
        def _():
            wait_store(ob1, os1)

        transpose_slab(st1, ob1)
        start_store(s0 + 1, ob1, os1)

    wait_store(ob0, os0)
    wait_store(ob1, os1)


def kernel(x, weights):
    out6 = _gather_kernel(weights, x.T)
    return out6.transpose(3, 5, 0, 1, 2, 4).reshape(
        BATCH, FIELD_NUM, FIELD_NUM, LATENT_NUM
    )


# speedup vs baseline: 2.1700x; 1.7627x over previous
import functools

import jax
import jax.numpy as jnp
from jax import lax
from jax.experimental import pallas as pl
from jax.experimental.pallas import tpu as pltpu
from jax.experimental.pallas import tpu_sc as plsc

FEATURE_NUM = 100000
FIELD_NUM = 26
LATENT_NUM = 32
BATCH = 4096

NC = 2
NS = 16
NW = NC * NS
BPW = BATCH // NW
QB = BPW // 4
LT = LATENT_NUM // 8
NSTEP = FIELD_NUM * 4
NPAIR = NSTEP // 2

_mesh = plsc.VectorSubcoreMesh(
    core_axis_name="c", subcore_axis_name="s", num_cores=NC, num_subcores=NS
)


@functools.partial(
    pl.kernel,
    out_type=jax.ShapeDtypeStruct(
        (FIELD_NUM, FIELD_NUM, LT, BATCH // 128, 8, 128), jnp.float32
    ),
    mesh=_mesh,
    scratch_types=[
        pltpu.VMEM((FIELD_NUM, BPW), jnp.int32),
        pltpu.VMEM((QB, FIELD_NUM, LATENT_NUM), jnp.float32),
        pltpu.VMEM((QB, FIELD_NUM, LATENT_NUM), jnp.float32),
        pltpu.VMEM((FIELD_NUM, LT, 8, QB + 1), jnp.float32),
        pltpu.VMEM((FIELD_NUM, LT, 8, QB + 1), jnp.float32),
        pltpu.SemaphoreType.DMA,
        pltpu.SemaphoreType.DMA,
        pltpu.SemaphoreType.DMA,
        pltpu.SemaphoreType.DMA,
    ],
    compiler_params=pltpu.CompilerParams(
        use_tc_tiling_on_sc=False, needs_layout_passes=False
    ),
)
def _gather_kernel(table_hbm, xt_hbm, out_hbm, idx_t, st0, st1, ob0, ob1,
                   g0, g1, os0, os1):
    wid = lax.axis_index("s") * NC + lax.axis_index("c")
    pltpu.sync_copy(xt_hbm.at[:, pl.ds(wid * BPW, BPW)], idx_t)

    iota = lax.iota(jnp.int32, 16)

    def start_gather(s, st, gsem):
        f1 = lax.shift_right_logical(s, 2)
        qoff = pl.multiple_of(lax.mul(lax.rem(s, 4), QB), QB)
        pltpu.async_copy(table_hbm.at[idx_t.at[f1, pl.ds(qoff, QB)]], st, gsem)

    def wait_gather(st, gsem):
        pltpu.make_async_copy(table_hbm.at[pl.ds(0, QB)], st, gsem).wait()

    def start_store(s, ob, osem):
        f1 = lax.shift_right_logical(s, 2)
        qoff = pl.multiple_of(lax.mul(lax.rem(s, 4), QB), QB)
        pltpu.async_copy(
            ob.at[:, :, :, pl.ds(0, QB)],
            out_hbm.at[f1, :, :, wid, :, pl.ds(qoff, QB)],
            osem,
        )

    def wait_store(ob, osem):
        pltpu.make_async_copy(
            ob.at[:, :, :, pl.ds(0, QB)],
            out_hbm.at[0, :, :, wid, :, pl.ds(0, QB)],
            osem,
        ).wait()

    ltv = {l0: lax.shift_right_logical(l0 + iota, 3) for l0 in (0, 16)}
    lsv = {l0: lax.bitwise_and(l0 + iota, 7) for l0 in (0, 16)}

    def transpose_slab(st, ob):
        pairs = [(b, l0) for l0 in (0, 16) for b in range(QB)]

        @pl.loop(0, FIELD_NUM)
        def f2loop(f2):
            f2v = jnp.full((16,), f2, jnp.int32)
            for w0 in range(0, len(pairs), 8):
                wave = pairs[w0:w0 + 8]
                vecs = [st[b, f2, pl.ds(l0, 16)] for (b, l0) in wave]
                for (b, l0), vec in zip(wave, vecs):
                    plsc.store_scatter(
                        ob,
                        [f2v, ltv[l0], lsv[l0], jnp.full((16,), b, jnp.int32)],
                        vec,
                    )

    start_gather(0, st0, g0)

    @pl.loop(0, NPAIR)
    def pair(p):
        s0 = p * 2

        start_gather(s0 + 1, st1, g1)
        wait_gather(st0, g0)

        @pl.when(p > 0)
        def _():
            wait_store(ob0, os0)

        transpose_slab(st0, ob0)
        start_store(s0, ob0, os0)

        @pl.when(p < NPAIR - 1)
        def _():
            start_gather(s0 + 2, st0, g0)

        wait_gather(st1, g1)

        @pl.when(p > 0)
        def _():
            wait_store(ob1, os1)

        transpose_slab(st1, ob1)
        start_store(s0 + 1, ob1, os1)

    wait_store(ob0, os0)
    wait_store(ob1, os1)


def kernel(x, weights):
    out6 = _gather_kernel(weights, x.T)
    return out6.transpose(3, 5, 0, 1, 2, 4).reshape(
        BATCH, FIELD_NUM, FIELD_NUM, LATENT_NUM
    )


# speedup vs baseline: 2.1704x; 1.0002x over previous
import functools

import jax
import jax.numpy as jnp
from jax import lax
from jax.experimental import pallas as pl
from jax.experimental.pallas import tpu as pltpu
from jax.experimental.pallas import tpu_sc as plsc

FEATURE_NUM = 100000
FIELD_NUM = 26
LATENT_NUM = 32
BATCH = 4096

NC = 2
NS = 16
NW = NC * NS
BPW = BATCH // NW
QB = BPW // 4
LT = LATENT_NUM // 8
NSTEP = FIELD_NUM * 4
NPAIR = NSTEP // 2

_mesh = plsc.VectorSubcoreMesh(
    core_axis_name="c", subcore_axis_name="s", num_cores=NC, num_subcores=NS
)


@functools.partial(
    pl.kernel,
    out_type=jax.ShapeDtypeStruct(
        (FIELD_NUM, FIELD_NUM, LT, BATCH // 128, 8, 128), jnp.float32
    ),
    mesh=_mesh,
    scratch_types=[
        pltpu.VMEM((FIELD_NUM, BPW), jnp.float32),
        pltpu.VMEM((FIELD_NUM, BPW), jnp.int32),
        pltpu.VMEM((QB, FIELD_NUM, LATENT_NUM), jnp.float32),
        pltpu.VMEM((QB, FIELD_NUM, LATENT_NUM), jnp.float32),
        pltpu.VMEM((FIELD_NUM, LT, 8, QB + 1), jnp.float32),
        pltpu.VMEM((FIELD_NUM, LT, 8, QB + 1), jnp.float32),
        pltpu.SemaphoreType.DMA,
        pltpu.SemaphoreType.DMA,
        pltpu.SemaphoreType.DMA,
        pltpu.SemaphoreType.DMA,
    ],
    compiler_params=pltpu.CompilerParams(
        use_tc_tiling_on_sc=False, needs_layout_passes=False
    ),
)
def _gather_kernel(table_hbm, xt_hbm, out_hbm, idxf, idx_t, st0, st1, ob0, ob1,
                   g0, g1, os0, os1):
    wid = lax.axis_index("s") * NC + lax.axis_index("c")
    pltpu.sync_copy(xt_hbm.at[:, pl.ds(wid * BPW, BPW)], idxf)
    for f1 in range(FIELD_NUM):
        for g in range(BPW // 16):
            idx_t[f1, pl.ds(g * 16, 16)] = plsc.bitcast(
                idxf[f1, pl.ds(g * 16, 16)], jnp.int32
            )

    iota = lax.iota(jnp.int32, 16)

    def start_gather(s, st, gsem):
        f1 = lax.shift_right_logical(s, 2)
        qoff = pl.multiple_of(lax.mul(lax.rem(s, 4), QB), QB)
        pltpu.async_copy(table_hbm.at[idx_t.at[f1, pl.ds(qoff, QB)]], st, gsem)

    def wait_gather(st, gsem):
        pltpu.make_async_copy(table_hbm.at[pl.ds(0, QB)], st, gsem).wait()

    def start_store(s, ob, osem):
        f1 = lax.shift_right_logical(s, 2)
        qoff = pl.multiple_of(lax.mul(lax.rem(s, 4), QB), QB)
        pltpu.async_copy(
            ob.at[:, :, :, pl.ds(0, QB)],
            out_hbm.at[f1, :, :, wid, :, pl.ds(qoff, QB)],
            osem,
        )

    def wait_store(ob, osem):
        pltpu.make_async_copy(
            ob.at[:, :, :, pl.ds(0, QB)],
            out_hbm.at[0, :, :, wid, :, pl.ds(0, QB)],
            osem,
        ).wait()

    ltv = {l0: lax.shift_right_logical(l0 + iota, 3) for l0 in (0, 16)}
    lsv = {l0: lax.bitwise_and(l0 + iota, 7) for l0 in (0, 16)}

    def transpose_slab(st, ob):
        pairs = [(b, l0) for l0 in (0, 16) for b in range(QB)]

        @pl.loop(0, FIELD_NUM)
        def f2loop(f2):
            f2v = jnp.full((16,), f2, jnp.int32)
            for w0 in range(0, len(pairs), 8):
                wave = pairs[w0:w0 + 8]
                vecs = [st[b, f2, pl.ds(l0, 16)] for (b, l0) in wave]
                for (b, l0), vec in zip(wave, vecs):
                    plsc.store_scatter(
                        ob,
                        [f2v, ltv[l0], lsv[l0], jnp.full((16,), b, jnp.int32)],
                        vec,
                    )

    start_gather(0, st0, g0)

    @pl.loop(0, NPAIR)
    def pair(p):
        s0 = p * 2

        start_gather(s0 + 1, st1, g1)
        wait_gather(st0, g0)

        @pl.when(p > 0)
        def _():
            wait_store(ob0, os0)

        transpose_slab(st0, ob0)
        start_store(s0, ob0, os0)

        @pl.when(p < NPAIR - 1)
        def _():
            start_gather(s0 + 2, st0, g0)

        wait_gather(st1, g1)

        @pl.when(p > 0)
        def _():
            wait_store(ob1, os1)

        transpose_slab(st1, ob1)
        start_store(s0 + 1, ob1, os1)

    wait_store(ob0, os0)
    wait_store(ob1, os1)


def kernel(x, weights):
    xf = jax.lax.bitcast_convert_type(x, jnp.float32)
    out6 = _gather_kernel(weights, xf.T)
    return out6.transpose(3, 5, 0, 1, 2, 4).reshape(
        BATCH, FIELD_NUM, FIELD_NUM, LATENT_NUM
    )
